# baseline (speedup 1.0000x reference)
import dataclasses
import functools

import jax
import jax.numpy as jnp
from jax import lax
from jax.experimental import pallas as pl
from jax.experimental.pallas import tpu as pltpu
from jax.experimental.pallas import tpu_sc as plsc

NC = 2
NS = 16
L = 16
NW = NC * NS
B = 16384
D = 32
ROWS_PER_TILE = 128 // D
TAB_ROWS = 1000000 // ROWS_PER_TILE
BPW = B // NW
CH = 128
NCHK = BPW // CH
CHG = CH // L
NG = BPW // L
NBUF = 3

_mesh = plsc.VectorSubcoreMesh(core_axis_name="c", subcore_axis_name="s")

_cp = pltpu.CompilerParams()
if "needs_layout_passes" in pltpu.CompilerParams.__dataclass_fields__:
    _cp = dataclasses.replace(_cp, needs_layout_passes=False)


def _gmf_body(uhi_hbm, ihi_hbm, uoff_hbm, ioff_hbm, utab_hbm, itab_hbm,
              w_hbm, b_hbm, out_hbm,
              uhi_v, ihi_v, uoff_v, ioff_v,
              ub0, ub1, ub2, ib0, ib1, ib2, wsc_v, b_v, o_v,
              sem0, sem1, sem2):
    wid = lax.axis_index("s") * NC + lax.axis_index("c")
    ubufs = (ub0, ub1, ub2)
    ibufs = (ib0, ib1, ib2)
    sems = (sem0, sem1, sem2)

    pltpu.sync_copy(uhi_hbm.at[wid], uhi_v)
    pltpu.sync_copy(ihi_hbm.at[wid], ihi_v)
    pltpu.sync_copy(uoff_hbm.at[wid], uoff_v)
    pltpu.sync_copy(ioff_hbm.at[wid], ioff_v)
    pltpu.sync_copy(w_hbm, wsc_v)
    pltpu.sync_copy(b_hbm, b_v)

    def fire(c):
        s = c % NBUF
        pltpu.async_copy(utab_hbm.at[uhi_v.at[c]], ubufs[s], sems[s])
        pltpu.async_copy(itab_hbm.at[ihi_v.at[c]], ibufs[s], sems[s])

    def drain(c):
        s = c % NBUF
        pltpu.make_async_copy(
            utab_hbm.at[pl.ds(0, CH)], ubufs[s], sems[s]).wait()
        pltpu.make_async_copy(
            itab_hbm.at[pl.ds(0, CH)], ibufs[s], sems[s]).wait()

    for c in range(min(NBUF, NCHK)):
        fire(c)

    for c in range(NCHK):
        s = c % NBUF
        drain(c)
        ubp, ibp = ubufs[s], ibufs[s]

        @pl.loop(0, CHG)
        def _(k):
            g = c * CHG + k
            rows = k * L + lax.iota(jnp.int32, L)
            uoffv = uoff_v.at[(g, pl.ds(0, L))][...]
            ioffv = ioff_v.at[(g, pl.ds(0, L))][...]
            acc = b_v[...]
            for d in range(D):
                dv = jnp.full((L,), d, jnp.int32)
                uv = plsc.load_gather(ubp, [rows, uoffv + dv])
                iv = plsc.load_gather(ibp, [rows, ioffv + dv])
                wv = wsc_v.at[(d, pl.ds(0, L))][...]
                acc = acc + uv * iv * wv
            o_v.at[(g, pl.ds(0, L))][...] = 1.0 / (1.0 + jnp.exp(-acc))

        if c + NBUF < NCHK:
            fire(c + NBUF)

    pltpu.sync_copy(o_v, out_hbm.at[wid])


@functools.partial(
    pl.kernel,
    out_type=jax.ShapeDtypeStruct((NW, NG, L), jnp.float32),
    mesh=_mesh,
    scratch_types=[
        pltpu.VMEM((NCHK, CH), jnp.int32),
        pltpu.VMEM((NCHK, CH), jnp.int32),
        pltpu.VMEM((NG, L), jnp.int32),
        pltpu.VMEM((NG, L), jnp.int32),
        pltpu.VMEM((CH, 128), jnp.float32),
        pltpu.VMEM((CH, 128), jnp.float32),
        pltpu.VMEM((CH, 128), jnp.float32),
        pltpu.VMEM((CH, 128), jnp.float32),
        pltpu.VMEM((CH, 128), jnp.float32),
        pltpu.VMEM((CH, 128), jnp.float32),
        pltpu.VMEM((D, L), jnp.float32),
        pltpu.VMEM((L,), jnp.float32),
        pltpu.VMEM((NG, L), jnp.float32),
        pltpu.SemaphoreType.DMA,
        pltpu.SemaphoreType.DMA,
        pltpu.SemaphoreType.DMA,
    ],
    compiler_params=_cp,
)
def _gmf_sc(*args):
    _gmf_body(*args)


@jax.jit
def kernel(user_ids, item_ids, user_table, item_table, W, b):
    uid = user_ids.astype(jnp.int32)
    iid = item_ids.astype(jnp.int32)
    uhi = (uid // ROWS_PER_TILE).reshape(NW, NCHK, CH)
    ihi = (iid // ROWS_PER_TILE).reshape(NW, NCHK, CH)
    uoff = ((uid % ROWS_PER_TILE) * D).reshape(NW, NG, L)
    ioff = ((iid % ROWS_PER_TILE) * D).reshape(NW, NG, L)
    utab = user_table.reshape(TAB_ROWS, 128)
    itab = item_table.reshape(TAB_ROWS, 128)
    w_bcast = jnp.broadcast_to(W.reshape(D, 1), (D, L)).astype(jnp.float32)
    b16 = jnp.full((L,), b[0], dtype=jnp.float32)
    out3 = _gmf_sc(uhi, ihi, uoff, ioff, utab, itab, w_bcast, b16)
    return out3.reshape(B)


# speedup vs baseline: 1.5062x; 1.5062x over previous
import dataclasses
import functools

import jax
import jax.numpy as jnp
from jax import lax
from jax.experimental import pallas as pl
from jax.experimental.pallas import tpu as pltpu
from jax.experimental.pallas import tpu_sc as plsc

NC = 2
NS = 16
L = 16
NW = NC * NS
B = 16384
D = 32
BPW = B // NW
CH = 128
NCHK = BPW // CH
CHG = CH // L
NG = BPW // L

_mesh = plsc.VectorSubcoreMesh(core_axis_name="c", subcore_axis_name="s")

_cp = pltpu.CompilerParams()
if "needs_layout_passes" in pltpu.CompilerParams.__dataclass_fields__:
    _cp = dataclasses.replace(_cp, needs_layout_passes=False)


def _gmf_body(uid_hbm, iid_hbm, utab_hbm, itab_hbm, w_hbm, b_hbm, out_hbm,
              usm, ism, uidx_v, iidx_v, ub0, ub1, ib0, ib1, wsc_v, b_v, o_v,
              idsem, sem0, sem1):
    wid = lax.axis_index("s") * NC + lax.axis_index("c")
    base = wid * BPW

    sid = lax.axis_index("s")
    pltpu.async_copy(uid_hbm.at[pl.ds(base, BPW)], uidx_v.at[sid], idsem).wait()
    pltpu.async_copy(iid_hbm.at[pl.ds(base, BPW)], iidx_v.at[sid], idsem).wait()
    pltpu.sync_copy(uidx_v.at[sid], usm)
    pltpu.sync_copy(iidx_v.at[sid], ism)
    pltpu.sync_copy(w_hbm, wsc_v)
    pltpu.sync_copy(b_hbm, b_v)

    ubufs = (ub0, ub1)
    ibufs = (ib0, ib1)
    sems = (sem0, sem1)

    def fire(c):
        p = c % 2
        ubp, ibp, sem = ubufs[p], ibufs[p], sems[p]

        @plsc.parallel_loop(0, CH, unroll=8)
        def _(r):
            j = c * CH + r
            pltpu.async_copy(utab_hbm.at[usm[j]], ubp.at[r], sem)
            pltpu.async_copy(itab_hbm.at[ism[j]], ibp.at[r], sem)

    def drain(c):
        p = c % 2
        pltpu.make_async_copy(
            utab_hbm.at[pl.ds(0, CH)], ubufs[p], sems[p]).wait()
        pltpu.make_async_copy(
            itab_hbm.at[pl.ds(0, CH)], ibufs[p], sems[p]).wait()

    fire(0)
    for c in range(NCHK):
        if c + 1 < NCHK:
            fire(c + 1)
        drain(c)
        ubp, ibp = ubufs[c % 2], ibufs[c % 2]

        @pl.loop(0, CHG)
        def _(k):
            g = c * CHG + k
            rows = k * L + lax.iota(jnp.int32, L)
            acc = b_v[...]
            for d in range(D):
                didx = jnp.full((L,), d, jnp.int32)
                uv = plsc.load_gather(ubp, [rows, didx])
                iv = plsc.load_gather(ibp, [rows, didx])
                wv = wsc_v.at[(d, pl.ds(0, L))][...]
                acc = acc + uv * iv * wv
            o_v.at[(g, pl.ds(0, L))][...] = 1.0 / (1.0 + jnp.exp(-acc))

    pltpu.sync_copy(o_v, out_hbm.at[wid])


@functools.partial(
    pl.kernel,
    out_type=jax.ShapeDtypeStruct((NW, NG, L), jnp.float32),
    mesh=_mesh,
    scratch_types=[
        pltpu.SMEM((BPW,), jnp.int32),
        pltpu.SMEM((BPW,), jnp.int32),
        pltpu.VMEM_SHARED((NS, BPW), jnp.int32),
        pltpu.VMEM_SHARED((NS, BPW), jnp.int32),
        pltpu.VMEM((CH, D), jnp.float32),
        pltpu.VMEM((CH, D), jnp.float32),
        pltpu.VMEM((CH, D), jnp.float32),
        pltpu.VMEM((CH, D), jnp.float32),
        pltpu.VMEM((D, L), jnp.float32),
        pltpu.VMEM((L,), jnp.float32),
        pltpu.VMEM((NG, L), jnp.float32),
        pltpu.SemaphoreType.DMA,
        pltpu.SemaphoreType.DMA,
        pltpu.SemaphoreType.DMA,
    ],
    compiler_params=_cp,
)
def _gmf_sc(*args):
    _gmf_body(*args)


@jax.jit
def kernel(user_ids, item_ids, user_table, item_table, W, b):
    uid = user_ids.astype(jnp.int32)
    iid = item_ids.astype(jnp.int32)
    w_bcast = jnp.broadcast_to(W.reshape(D, 1), (D, L)).astype(jnp.float32)
    b16 = jnp.full((L,), b[0], dtype=jnp.float32)
    out3 = _gmf_sc(uid, iid, user_table, item_table, w_bcast, b16)
    return out3.reshape(B)
